# baseline (speedup 1.0000x reference)
import jax
import jax.numpy as jnp
from jax.experimental import pallas as pl
from jax.experimental.pallas import tpu as pltpu

_TB = 1024


def _mlp3_kernel(x_ref, w1_ref, b1_ref, w2_ref, b2_ref, w3_ref, b3_ref,
                 o_ref):
    acc = x_ref[...]
    layers = ((w1_ref, b1_ref, True),
              (w2_ref, b2_ref, True),
              (w3_ref, b3_ref, False))
    for w_ref, b_ref, relu in layers:
        acc = jnp.dot(acc, w_ref[...],
                      preferred_element_type=jnp.float32) + b_ref[...]
        if relu:
            acc = jnp.maximum(acc, 0.0)
    o_ref[...] = acc.astype(o_ref.dtype)


def _full(shape):
    return pl.BlockSpec(shape, lambda i: (0,) * len(shape))


def kernel(x, w1, b1, w2, b2, w3, b3):
    b, e = x.shape
    h = w1.shape[1]
    c = w3.shape[1]

    tb = _TB if b % _TB == 0 else 8
    grid = (b // tb,)

    return pl.pallas_call(
        _mlp3_kernel,
        out_shape=jax.ShapeDtypeStruct((b, c), x.dtype),
        grid=grid,
        in_specs=[
            pl.BlockSpec((tb, e), lambda i: (i, 0)),
            _full((e, h)),
            _full((1, h)),
            _full((h, h)),
            _full((1, h)),
            _full((h, c)),
            _full((1, c)),
        ],
        out_specs=pl.BlockSpec((tb, c), lambda i: (i, 0)),
        compiler_params=pltpu.CompilerParams(
            dimension_semantics=("parallel",),
            vmem_limit_bytes=int(60 << 20),
        ),
    )(x, w1, b1, w2, b2, w3, b3)
